# baseline (speedup 1.0000x reference)
import functools

import jax
import jax.numpy as jnp
from jax import lax
from jax.experimental import pallas as pl
from jax.experimental.pallas import tpu as pltpu
from jax.experimental.pallas import tpu_sc as plsc

B = 4096
D = 1024
BR = 256
CT = 512
NRB = B // BR
NCT = B // CT
TOPK = 10
IDX_COLS = 16
NEG = -3.0e38

NC = 2
NS = 16
NW = NC * NS
RPW = B // NW


def _dense_body(scale_ref, img_ref, txt_r_ref, txt_c_ref,
                l_ref, lse_row_ref, lse_col_ref, idx_ref,
                cand_scr, pp_scr, rmax_scr, rsum_scr, cmax_scr, csum_scr):
    rb = pl.program_id(0)
    ct = pl.program_id(1)
    scale = scale_ref[0, 0]

    img = img_ref[...]
    txt_r = txt_r_ref[...]
    txt_c = txt_c_ref[...]

    l_tile = scale * lax.dot_general(
        img.astype(jnp.bfloat16), txt_c.astype(jnp.bfloat16),
        (((1,), (1,)), ((), ())),
        preferred_element_type=jnp.float32)
    l_ref[...] = l_tile

    inv_r = 1.0 / jnp.maximum(
        jnp.sqrt(jnp.sum(txt_r * txt_r, axis=1, keepdims=True)), 1e-12)
    inv_c = 1.0 / jnp.maximum(
        jnp.sqrt(jnp.sum(txt_c * txt_c, axis=1, keepdims=True)), 1e-12)
    sim_tile = lax.dot_general(
        (txt_r * inv_r).astype(jnp.bfloat16),
        (txt_c * inv_c).astype(jnp.bfloat16),
        (((1,), (1,)), ((), ())),
        preferred_element_type=jnp.float32)

    kmin = jnp.int32(-2147483648)

    def topk_chunk(sim_chunk, chunk_id):
        col_g = chunk_id * CT + lax.broadcasted_iota(jnp.int32, (BR, CT), 1)
        row_g = rb * BR + lax.broadcasted_iota(jnp.int32, (BR, CT), 0)
        w = jnp.where(col_g == row_g, 0.0, sim_chunk)
        u = lax.bitcast_convert_type(w, jnp.int32)
        ki = jnp.where(w < 0, u ^ jnp.int32(0x7FFFFFFF), u)
        key = (ki & jnp.int32(-4096)) | ((B - 1) - col_g)
        for m in range(TOPK):
            mx = jnp.max(key, axis=1, keepdims=True)
            cand_scr[chunk_id, :, m:m + 1] = mx
            key = jnp.where(key == mx, kmin, key)
        cand_scr[chunk_id, :, TOPK:] = jnp.full(
            (BR, IDX_COLS - TOPK), kmin, jnp.int32)

    pp_scr[ct % 2] = sim_tile

    @pl.when(ct > 0)
    def _():
        topk_chunk(pp_scr[1 - ct % 2], ct - 1)

    t_rmax = jnp.max(l_tile, axis=1, keepdims=True)

    @pl.when(ct == 0)
    def _():
        rmax_scr[:, 0:1] = t_rmax
        rsum_scr[:, 0:1] = jnp.sum(jnp.exp(l_tile - t_rmax), axis=1,
                                   keepdims=True)

    @pl.when(ct > 0)
    def _():
        om = rmax_scr[:, 0:1]
        nm = jnp.maximum(om, t_rmax)
        rsum_scr[:, 0:1] = (rsum_scr[:, 0:1] * jnp.exp(om - nm)
                            + jnp.sum(jnp.exp(l_tile - nm), axis=1,
                                      keepdims=True))
        rmax_scr[:, 0:1] = nm

    t_cmax = jnp.max(l_tile, axis=0, keepdims=True)

    @pl.when(rb == 0)
    def _():
        cmax_scr[0:1, pl.ds(ct * CT, CT)] = t_cmax
        csum_scr[0:1, pl.ds(ct * CT, CT)] = jnp.sum(
            jnp.exp(l_tile - t_cmax), axis=0, keepdims=True)

    @pl.when(rb > 0)
    def _():
        om = cmax_scr[0:1, pl.ds(ct * CT, CT)]
        nm = jnp.maximum(om, t_cmax)
        csum_scr[0:1, pl.ds(ct * CT, CT)] = (
            csum_scr[0:1, pl.ds(ct * CT, CT)] * jnp.exp(om - nm)
            + jnp.sum(jnp.exp(l_tile - nm), axis=0, keepdims=True))
        cmax_scr[0:1, pl.ds(ct * CT, CT)] = nm

    @pl.when(ct == NCT - 1)
    def _():
        topk_chunk(sim_tile, NCT - 1)
        mk = jnp.concatenate([cand_scr[c] for c in range(NCT)], axis=1)
        for m in range(TOPK):
            mx = jnp.max(mk, axis=1, keepdims=True)
            idx_ref[:, m:m + 1] = (B - 1) - (mx & jnp.int32(4095))
            mk = jnp.where(mk == mx, jnp.int32(-2147483648), mk)
        own = rb * BR + lax.broadcasted_iota(jnp.int32, (BR, 1), 0)
        for m in range(TOPK, IDX_COLS):
            idx_ref[:, m:m + 1] = own
        lse_row_ref[...] = rmax_scr[:, 0:1] + jnp.log(rsum_scr[:, 0:1])

    @pl.when((rb == NRB - 1) & (ct == NCT - 1))
    def _():
        lse_col_ref[...] = cmax_scr[0:1, :] + jnp.log(csum_scr[0:1, :])


_dense_call = pl.pallas_call(
    _dense_body,
    grid=(NRB, NCT),
    in_specs=[
        pl.BlockSpec((1, 1), lambda rb, ct: (0, 0),
                     memory_space=pltpu.SMEM),
        pl.BlockSpec((BR, D), lambda rb, ct: (rb, 0)),
        pl.BlockSpec((BR, D), lambda rb, ct: (rb, 0)),
        pl.BlockSpec((CT, D), lambda rb, ct: (ct, 0)),
    ],
    out_specs=[
        pl.BlockSpec((BR, CT), lambda rb, ct: (rb, ct)),
        pl.BlockSpec((BR, 1), lambda rb, ct: (rb, 0)),
        pl.BlockSpec((1, B), lambda rb, ct: (0, 0)),
        pl.BlockSpec((BR, IDX_COLS), lambda rb, ct: (rb, 0)),
    ],
    out_shape=[
        jax.ShapeDtypeStruct((B, B), jnp.float32),
        jax.ShapeDtypeStruct((B, 1), jnp.float32),
        jax.ShapeDtypeStruct((1, B), jnp.float32),
        jax.ShapeDtypeStruct((B, IDX_COLS), jnp.int32),
    ],
    scratch_shapes=[
        pltpu.VMEM((NCT, BR, IDX_COLS), jnp.int32),
        pltpu.VMEM((2, BR, CT), jnp.float32),
        pltpu.VMEM((BR, 128), jnp.float32),
        pltpu.VMEM((BR, 128), jnp.float32),
        pltpu.VMEM((8, B), jnp.float32),
        pltpu.VMEM((8, B), jnp.float32),
    ],
    compiler_params=pltpu.CompilerParams(
        dimension_semantics=("arbitrary", "arbitrary")),
)


def _sc_body(lflat_hbm, idx_hbm, cls_hbm, aimg_hbm, atxt_hbm,
             cls_v, idx_v, fimg_v, ftxt_v, vimg_v, vtxt_v, wgt_v,
             oimg_v, otxt_v, sem):
    wid = lax.axis_index("s") * NC + lax.axis_index("c")
    base = wid * RPW
    pltpu.sync_copy(cls_hbm, cls_v)
    pltpu.sync_copy(idx_hbm.at[pl.ds(base, RPW)], idx_v)

    lane = lax.iota(jnp.int32, 16)

    def build(jl, carry):
        j = base + jl
        jv = jnp.broadcast_to(j, (16,))
        iv = idx_v[jl, :]
        iv = jnp.where(lane >= TOPK, jv, iv)
        cls_i = plsc.load_gather(cls_v, [iv])
        cls_j = plsc.load_gather(cls_v, [jv])
        keep = (cls_i == cls_j) & (iv != jv) & (lane < TOPK)
        keep = keep | (lane == TOPK)
        wgt_v[pl.ds(jl * IDX_COLS, IDX_COLS)] = jnp.where(keep, 1.0, 0.0)
        fimg_v[pl.ds(jl * IDX_COLS, IDX_COLS)] = jv * B + iv
        ftxt_v[pl.ds(jl * IDX_COLS, IDX_COLS)] = iv * B + jv
        return carry

    lax.fori_loop(0, RPW, build, 0)

    copies = []
    for c in range(RPW * IDX_COLS // 128):
        sl = pl.ds(c * 128, 128)
        copies.append(pltpu.async_copy(
            lflat_hbm.at[fimg_v.at[sl]], vimg_v.at[sl], sem))
        copies.append(pltpu.async_copy(
            lflat_hbm.at[ftxt_v.at[sl]], vtxt_v.at[sl], sem))
    for cp in copies:
        cp.wait()

    def accum(c, carry):
        rows16 = c * IDX_COLS + lane
        gdiag = rows16 * IDX_COLS + TOPK
        acc_i = plsc.load_gather(vimg_v, [gdiag])
        acc_t = plsc.load_gather(vtxt_v, [gdiag])
        s = jnp.full((16,), 1.0, jnp.float32)
        for m in range(TOPK):
            g = rows16 * IDX_COLS + m
            w = plsc.load_gather(wgt_v, [g])
            s = s + w
            acc_i = acc_i + w * plsc.load_gather(vimg_v, [g])
            acc_t = acc_t + w * plsc.load_gather(vtxt_v, [g])
        oimg_v[pl.ds(c * IDX_COLS, IDX_COLS)] = acc_i / s
        otxt_v[pl.ds(c * IDX_COLS, IDX_COLS)] = acc_t / s
        return carry

    lax.fori_loop(0, RPW // IDX_COLS, accum, 0)

    pltpu.sync_copy(oimg_v, aimg_hbm.at[pl.ds(base, RPW)])
    pltpu.sync_copy(otxt_v, atxt_hbm.at[pl.ds(base, RPW)])


@functools.cache
def _sc_call():
    return functools.partial(
        pl.kernel,
        mesh=plsc.VectorSubcoreMesh(core_axis_name="c", subcore_axis_name="s"),
        compiler_params=pltpu.CompilerParams(needs_layout_passes=False),
        out_type=[
            jax.ShapeDtypeStruct((B,), jnp.float32),
            jax.ShapeDtypeStruct((B,), jnp.float32),
        ],
        scratch_types=[
            pltpu.VMEM((B,), jnp.int32),
            pltpu.VMEM((RPW, IDX_COLS), jnp.int32),
            pltpu.VMEM((RPW * IDX_COLS,), jnp.int32),
            pltpu.VMEM((RPW * IDX_COLS,), jnp.int32),
            pltpu.VMEM((RPW * IDX_COLS,), jnp.float32),
            pltpu.VMEM((RPW * IDX_COLS,), jnp.float32),
            pltpu.VMEM((RPW * IDX_COLS,), jnp.float32),
            pltpu.VMEM((RPW,), jnp.float32),
            pltpu.VMEM((RPW,), jnp.float32),
            pltpu.SemaphoreType.DMA,
        ],
    )(_sc_body)


def kernel(image_features, text_features, logit_scale, img_index):
    scale2d = jnp.asarray(logit_scale, jnp.float32).reshape(1, 1)
    logits, lse_row, lse_col, idx = _dense_call(
        scale2d, image_features, text_features, text_features)
    a_img, a_txt = _sc_call()(logits.reshape(-1), idx, img_index)
    image_loss = jnp.mean(lse_row[:, 0] - a_img)
    text_loss = jnp.mean(lse_col[0, :] - a_txt)
    return (image_loss, text_loss, logits)


# speedup vs baseline: 1.6090x; 1.6090x over previous
import functools

import jax
import jax.numpy as jnp
from jax import lax
from jax.experimental import pallas as pl
from jax.experimental.pallas import tpu as pltpu
from jax.experimental.pallas import tpu_sc as plsc

B = 4096
D = 1024
BR = 512
CT = 512
NRB = B // BR
NCT = B // CT
TOPK = 10
IDX_COLS = 16
NEG = -3.0e38

NC = 2
NS = 16
NW = NC * NS
RPW = B // NW


def _dense_body(scale_ref, img_ref, txt_r_ref, txt_c_ref,
                l_ref, lse_row_ref, lse_col_ref, idx_ref,
                sim_scr, rmax_scr, rsum_scr, cmax_scr, csum_scr):
    rb = pl.program_id(0)
    ct = pl.program_id(1)
    scale = scale_ref[0, 0]

    img = img_ref[...]
    txt_r = txt_r_ref[...]
    txt_c = txt_c_ref[...]

    l_tile = scale * lax.dot_general(
        img.astype(jnp.bfloat16), txt_c.astype(jnp.bfloat16),
        (((1,), (1,)), ((), ())),
        preferred_element_type=jnp.float32)
    l_ref[...] = l_tile

    inv_r = 1.0 / jnp.maximum(
        jnp.sqrt(jnp.sum(txt_r * txt_r, axis=1, keepdims=True)), 1e-12)
    inv_c = 1.0 / jnp.maximum(
        jnp.sqrt(jnp.sum(txt_c * txt_c, axis=1, keepdims=True)), 1e-12)
    sim_tile = lax.dot_general(
        (txt_r * inv_r).astype(jnp.bfloat16),
        (txt_c * inv_c).astype(jnp.bfloat16),
        (((1,), (1,)), ((), ())),
        preferred_element_type=jnp.float32)

    sim_scr[:, pl.ds(ct * CT, CT)] = sim_tile

    t_rmax = jnp.max(l_tile, axis=1, keepdims=True)

    @pl.when(ct == 0)
    def _():
        rmax_scr[:, 0:1] = t_rmax
        rsum_scr[:, 0:1] = jnp.sum(jnp.exp(l_tile - t_rmax), axis=1,
                                   keepdims=True)

    @pl.when(ct > 0)
    def _():
        om = rmax_scr[:, 0:1]
        nm = jnp.maximum(om, t_rmax)
        rsum_scr[:, 0:1] = (rsum_scr[:, 0:1] * jnp.exp(om - nm)
                            + jnp.sum(jnp.exp(l_tile - nm), axis=1,
                                      keepdims=True))
        rmax_scr[:, 0:1] = nm

    t_cmax = jnp.max(l_tile, axis=0, keepdims=True)

    @pl.when(rb == 0)
    def _():
        cmax_scr[0:1, pl.ds(ct * CT, CT)] = t_cmax
        csum_scr[0:1, pl.ds(ct * CT, CT)] = jnp.sum(
            jnp.exp(l_tile - t_cmax), axis=0, keepdims=True)

    @pl.when(rb > 0)
    def _():
        om = cmax_scr[0:1, pl.ds(ct * CT, CT)]
        nm = jnp.maximum(om, t_cmax)
        csum_scr[0:1, pl.ds(ct * CT, CT)] = (
            csum_scr[0:1, pl.ds(ct * CT, CT)] * jnp.exp(om - nm)
            + jnp.sum(jnp.exp(l_tile - nm), axis=0, keepdims=True))
        cmax_scr[0:1, pl.ds(ct * CT, CT)] = nm

    @pl.when(ct == NCT - 1)
    def _():
        col_iota = lax.broadcasted_iota(jnp.int32, (BR, B), 1)
        row_ids = rb * BR + lax.broadcasted_iota(jnp.int32, (BR, B), 0)
        w = jnp.where(col_iota == row_ids, 0.0, sim_scr[...])
        u = lax.bitcast_convert_type(w, jnp.int32)
        ki = jnp.where(w < 0, u ^ jnp.int32(0x7FFFFFFF), u)
        mk = (ki & jnp.int32(-4096)) | ((B - 1) - col_iota)
        for m in range(TOPK):
            mx = jnp.max(mk, axis=1, keepdims=True)
            idx_ref[:, m:m + 1] = (B - 1) - (mx & jnp.int32(4095))
            mk = jnp.where(mk == mx, jnp.int32(-2147483648), mk)
        own = rb * BR + lax.broadcasted_iota(jnp.int32, (BR, 1), 0)
        for m in range(TOPK, IDX_COLS):
            idx_ref[:, m:m + 1] = own
        lse_row_ref[...] = rmax_scr[:, 0:1] + jnp.log(rsum_scr[:, 0:1])

    @pl.when((rb == NRB - 1) & (ct == NCT - 1))
    def _():
        lse_col_ref[...] = cmax_scr[0:1, :] + jnp.log(csum_scr[0:1, :])


_dense_call = pl.pallas_call(
    _dense_body,
    grid=(NRB, NCT),
    in_specs=[
        pl.BlockSpec((1, 1), lambda rb, ct: (0, 0),
                     memory_space=pltpu.SMEM),
        pl.BlockSpec((BR, D), lambda rb, ct: (rb, 0)),
        pl.BlockSpec((BR, D), lambda rb, ct: (rb, 0)),
        pl.BlockSpec((CT, D), lambda rb, ct: (ct, 0)),
    ],
    out_specs=[
        pl.BlockSpec((BR, CT), lambda rb, ct: (rb, ct)),
        pl.BlockSpec((BR, 1), lambda rb, ct: (rb, 0)),
        pl.BlockSpec((1, B), lambda rb, ct: (0, 0)),
        pl.BlockSpec((BR, IDX_COLS), lambda rb, ct: (rb, 0)),
    ],
    out_shape=[
        jax.ShapeDtypeStruct((B, B), jnp.float32),
        jax.ShapeDtypeStruct((B, 1), jnp.float32),
        jax.ShapeDtypeStruct((1, B), jnp.float32),
        jax.ShapeDtypeStruct((B, IDX_COLS), jnp.int32),
    ],
    scratch_shapes=[
        pltpu.VMEM((BR, B), jnp.float32),
        pltpu.VMEM((BR, 128), jnp.float32),
        pltpu.VMEM((BR, 128), jnp.float32),
        pltpu.VMEM((8, B), jnp.float32),
        pltpu.VMEM((8, B), jnp.float32),
    ],
    compiler_params=pltpu.CompilerParams(
        dimension_semantics=("arbitrary", "arbitrary")),
)


def _sc_body(lflat_hbm, idx_hbm, cls_hbm, aimg_hbm, atxt_hbm,
             cls_v, idx_v, fimg_v, ftxt_v, vimg_v, vtxt_v, wgt_v,
             oimg_v, otxt_v, sem):
    wid = lax.axis_index("s") * NC + lax.axis_index("c")
    base = wid * RPW
    pltpu.sync_copy(cls_hbm, cls_v)
    pltpu.sync_copy(idx_hbm.at[pl.ds(base, RPW)], idx_v)

    lane = lax.iota(jnp.int32, 16)

    def build(jl, carry):
        j = base + jl
        jv = jnp.broadcast_to(j, (16,))
        iv = idx_v[jl, :]
        iv = jnp.where(lane >= TOPK, jv, iv)
        cls_i = plsc.load_gather(cls_v, [iv])
        cls_j = plsc.load_gather(cls_v, [jv])
        keep = (cls_i == cls_j) & (iv != jv) & (lane < TOPK)
        keep = keep | (lane == TOPK)
        wgt_v[pl.ds(jl * IDX_COLS, IDX_COLS)] = jnp.where(keep, 1.0, 0.0)
        fimg_v[pl.ds(jl * IDX_COLS, IDX_COLS)] = jv * B + iv
        ftxt_v[pl.ds(jl * IDX_COLS, IDX_COLS)] = iv * B + jv
        return carry

    lax.fori_loop(0, RPW, build, 0)

    copies = []
    for c in range(RPW * IDX_COLS // 128):
        sl = pl.ds(c * 128, 128)
        copies.append(pltpu.async_copy(
            lflat_hbm.at[fimg_v.at[sl]], vimg_v.at[sl], sem))
        copies.append(pltpu.async_copy(
            lflat_hbm.at[ftxt_v.at[sl]], vtxt_v.at[sl], sem))
    for cp in copies:
        cp.wait()

    def accum(c, carry):
        rows16 = c * IDX_COLS + lane
        gdiag = rows16 * IDX_COLS + TOPK
        acc_i = plsc.load_gather(vimg_v, [gdiag])
        acc_t = plsc.load_gather(vtxt_v, [gdiag])
        s = jnp.full((16,), 1.0, jnp.float32)
        for m in range(TOPK):
            g = rows16 * IDX_COLS + m
            w = plsc.load_gather(wgt_v, [g])
            s = s + w
            acc_i = acc_i + w * plsc.load_gather(vimg_v, [g])
            acc_t = acc_t + w * plsc.load_gather(vtxt_v, [g])
        oimg_v[pl.ds(c * IDX_COLS, IDX_COLS)] = acc_i / s
        otxt_v[pl.ds(c * IDX_COLS, IDX_COLS)] = acc_t / s
        return carry

    lax.fori_loop(0, RPW // IDX_COLS, accum, 0)

    pltpu.sync_copy(oimg_v, aimg_hbm.at[pl.ds(base, RPW)])
    pltpu.sync_copy(otxt_v, atxt_hbm.at[pl.ds(base, RPW)])


@functools.cache
def _sc_call():
    return functools.partial(
        pl.kernel,
        mesh=plsc.VectorSubcoreMesh(core_axis_name="c", subcore_axis_name="s"),
        compiler_params=pltpu.CompilerParams(needs_layout_passes=False),
        out_type=[
            jax.ShapeDtypeStruct((B,), jnp.float32),
            jax.ShapeDtypeStruct((B,), jnp.float32),
        ],
        scratch_types=[
            pltpu.VMEM((B,), jnp.int32),
            pltpu.VMEM((RPW, IDX_COLS), jnp.int32),
            pltpu.VMEM((RPW * IDX_COLS,), jnp.int32),
            pltpu.VMEM((RPW * IDX_COLS,), jnp.int32),
            pltpu.VMEM((RPW * IDX_COLS,), jnp.float32),
            pltpu.VMEM((RPW * IDX_COLS,), jnp.float32),
            pltpu.VMEM((RPW * IDX_COLS,), jnp.float32),
            pltpu.VMEM((RPW,), jnp.float32),
            pltpu.VMEM((RPW,), jnp.float32),
            pltpu.SemaphoreType.DMA,
        ],
    )(_sc_body)


def kernel(image_features, text_features, logit_scale, img_index):
    scale2d = jnp.asarray(logit_scale, jnp.float32).reshape(1, 1)
    logits, lse_row, lse_col, idx = _dense_call(
        scale2d, image_features, text_features, text_features)
    a_img, a_txt = _sc_call()(logits.reshape(-1), idx, img_index)
    image_loss = jnp.mean(lse_row[:, 0] - a_img)
    text_loss = jnp.mean(lse_col[0, :] - a_txt)
    return (image_loss, text_loss, logits)


# speedup vs baseline: 1.7165x; 1.0668x over previous
import functools

import jax
import jax.numpy as jnp
from jax import lax
from jax.experimental import pallas as pl
from jax.experimental.pallas import tpu as pltpu
from jax.experimental.pallas import tpu_sc as plsc

B = 4096
D = 1024
BR = 512
CT = 1024
NRB = B // BR
NCT = B // CT
TOPK = 10
IDX_COLS = 16
NEG = -3.0e38

NC = 2
NS = 16
NW = NC * NS
RPW = B // NW


def _dense_body(scale_ref, img_ref, txt_r_ref, txt_c_ref,
                l_ref, lse_row_ref, lse_col_ref, idx_ref,
                sim_scr, rmax_scr, rsum_scr, cmax_scr, csum_scr):
    rb = pl.program_id(0)
    ct = pl.program_id(1)
    scale = scale_ref[0, 0]

    img = img_ref[...]
    txt_r = txt_r_ref[...]
    txt_c = txt_c_ref[...]

    l_tile = scale * lax.dot_general(
        img.astype(jnp.bfloat16), txt_c.astype(jnp.bfloat16),
        (((1,), (1,)), ((), ())),
        preferred_element_type=jnp.float32)
    l_ref[...] = l_tile

    inv_r = 1.0 / jnp.maximum(
        jnp.sqrt(jnp.sum(txt_r * txt_r, axis=1, keepdims=True)), 1e-12)
    inv_c = 1.0 / jnp.maximum(
        jnp.sqrt(jnp.sum(txt_c * txt_c, axis=1, keepdims=True)), 1e-12)
    sim_tile = lax.dot_general(
        (txt_r * inv_r).astype(jnp.bfloat16),
        (txt_c * inv_c).astype(jnp.bfloat16),
        (((1,), (1,)), ((), ())),
        preferred_element_type=jnp.float32)

    sim_scr[:, pl.ds(ct * CT, CT)] = sim_tile

    t_rmax = jnp.max(l_tile, axis=1, keepdims=True)

    @pl.when(ct == 0)
    def _():
        rmax_scr[:, 0:1] = t_rmax
        rsum_scr[:, 0:1] = jnp.sum(jnp.exp(l_tile - t_rmax), axis=1,
                                   keepdims=True)

    @pl.when(ct > 0)
    def _():
        om = rmax_scr[:, 0:1]
        nm = jnp.maximum(om, t_rmax)
        rsum_scr[:, 0:1] = (rsum_scr[:, 0:1] * jnp.exp(om - nm)
                            + jnp.sum(jnp.exp(l_tile - nm), axis=1,
                                      keepdims=True))
        rmax_scr[:, 0:1] = nm

    t_cmax = jnp.max(l_tile, axis=0, keepdims=True)

    @pl.when(rb == 0)
    def _():
        cmax_scr[0:1, pl.ds(ct * CT, CT)] = t_cmax
        csum_scr[0:1, pl.ds(ct * CT, CT)] = jnp.sum(
            jnp.exp(l_tile - t_cmax), axis=0, keepdims=True)

    @pl.when(rb > 0)
    def _():
        om = cmax_scr[0:1, pl.ds(ct * CT, CT)]
        nm = jnp.maximum(om, t_cmax)
        csum_scr[0:1, pl.ds(ct * CT, CT)] = (
            csum_scr[0:1, pl.ds(ct * CT, CT)] * jnp.exp(om - nm)
            + jnp.sum(jnp.exp(l_tile - nm), axis=0, keepdims=True))
        cmax_scr[0:1, pl.ds(ct * CT, CT)] = nm

    @pl.when(ct == NCT - 1)
    def _():
        col_iota = lax.broadcasted_iota(jnp.int32, (BR, B), 1)
        row_ids = rb * BR + lax.broadcasted_iota(jnp.int32, (BR, B), 0)
        w = jnp.where(col_iota == row_ids, 0.0, sim_scr[...])
        u = lax.bitcast_convert_type(w, jnp.int32)
        ki = jnp.where(w < 0, u ^ jnp.int32(0x7FFFFFFF), u)
        mk = (ki & jnp.int32(-4096)) | ((B - 1) - col_iota)
        for m in range(TOPK):
            mx = jnp.max(mk, axis=1, keepdims=True)
            idx_ref[:, m:m + 1] = (B - 1) - (mx & jnp.int32(4095))
            mk = jnp.where(mk == mx, jnp.int32(-2147483648), mk)
        own = rb * BR + lax.broadcasted_iota(jnp.int32, (BR, 1), 0)
        for m in range(TOPK, IDX_COLS):
            idx_ref[:, m:m + 1] = own
        lse_row_ref[...] = rmax_scr[:, 0:1] + jnp.log(rsum_scr[:, 0:1])

    @pl.when((rb == NRB - 1) & (ct == NCT - 1))
    def _():
        lse_col_ref[...] = cmax_scr[0:1, :] + jnp.log(csum_scr[0:1, :])


_dense_call = pl.pallas_call(
    _dense_body,
    grid=(NRB, NCT),
    in_specs=[
        pl.BlockSpec((1, 1), lambda rb, ct: (0, 0),
                     memory_space=pltpu.SMEM),
        pl.BlockSpec((BR, D), lambda rb, ct: (rb, 0)),
        pl.BlockSpec((BR, D), lambda rb, ct: (rb, 0)),
        pl.BlockSpec((CT, D), lambda rb, ct: (ct, 0)),
    ],
    out_specs=[
        pl.BlockSpec((BR, CT), lambda rb, ct: (rb, ct)),
        pl.BlockSpec((BR, 1), lambda rb, ct: (rb, 0)),
        pl.BlockSpec((1, B), lambda rb, ct: (0, 0)),
        pl.BlockSpec((BR, IDX_COLS), lambda rb, ct: (rb, 0)),
    ],
    out_shape=[
        jax.ShapeDtypeStruct((B, B), jnp.float32),
        jax.ShapeDtypeStruct((B, 1), jnp.float32),
        jax.ShapeDtypeStruct((1, B), jnp.float32),
        jax.ShapeDtypeStruct((B, IDX_COLS), jnp.int32),
    ],
    scratch_shapes=[
        pltpu.VMEM((BR, B), jnp.float32),
        pltpu.VMEM((BR, 128), jnp.float32),
        pltpu.VMEM((BR, 128), jnp.float32),
        pltpu.VMEM((8, B), jnp.float32),
        pltpu.VMEM((8, B), jnp.float32),
    ],
    compiler_params=pltpu.CompilerParams(
        dimension_semantics=("arbitrary", "arbitrary")),
)


def _sc_body(lflat_hbm, idx_hbm, cls_hbm, aimg_hbm, atxt_hbm,
             cls_v, idx_v, fimg_v, ftxt_v, vimg_v, vtxt_v, wgt_v,
             oimg_v, otxt_v, sem):
    wid = lax.axis_index("s") * NC + lax.axis_index("c")
    base = wid * RPW
    pltpu.sync_copy(cls_hbm, cls_v)
    pltpu.sync_copy(idx_hbm.at[pl.ds(base, RPW)], idx_v)

    lane = lax.iota(jnp.int32, 16)

    def build(jl, carry):
        j = base + jl
        jv = jnp.broadcast_to(j, (16,))
        iv = idx_v[jl, :]
        iv = jnp.where(lane >= TOPK, jv, iv)
        cls_i = plsc.load_gather(cls_v, [iv])
        cls_j = plsc.load_gather(cls_v, [jv])
        keep = (cls_i == cls_j) & (iv != jv) & (lane < TOPK)
        keep = keep | (lane == TOPK)
        wgt_v[pl.ds(jl * IDX_COLS, IDX_COLS)] = jnp.where(keep, 1.0, 0.0)
        fimg_v[pl.ds(jl * IDX_COLS, IDX_COLS)] = jv * B + iv
        ftxt_v[pl.ds(jl * IDX_COLS, IDX_COLS)] = iv * B + jv
        return carry

    lax.fori_loop(0, RPW, build, 0)

    copies = []
    for c in range(RPW * IDX_COLS // 128):
        sl = pl.ds(c * 128, 128)
        copies.append(pltpu.async_copy(
            lflat_hbm.at[fimg_v.at[sl]], vimg_v.at[sl], sem))
        copies.append(pltpu.async_copy(
            lflat_hbm.at[ftxt_v.at[sl]], vtxt_v.at[sl], sem))
    for cp in copies:
        cp.wait()

    def accum(c, carry):
        rows16 = c * IDX_COLS + lane
        gdiag = rows16 * IDX_COLS + TOPK
        acc_i = plsc.load_gather(vimg_v, [gdiag])
        acc_t = plsc.load_gather(vtxt_v, [gdiag])
        s = jnp.full((16,), 1.0, jnp.float32)
        for m in range(TOPK):
            g = rows16 * IDX_COLS + m
            w = plsc.load_gather(wgt_v, [g])
            s = s + w
            acc_i = acc_i + w * plsc.load_gather(vimg_v, [g])
            acc_t = acc_t + w * plsc.load_gather(vtxt_v, [g])
        oimg_v[pl.ds(c * IDX_COLS, IDX_COLS)] = acc_i / s
        otxt_v[pl.ds(c * IDX_COLS, IDX_COLS)] = acc_t / s
        return carry

    lax.fori_loop(0, RPW // IDX_COLS, accum, 0)

    pltpu.sync_copy(oimg_v, aimg_hbm.at[pl.ds(base, RPW)])
    pltpu.sync_copy(otxt_v, atxt_hbm.at[pl.ds(base, RPW)])


@functools.cache
def _sc_call():
    return functools.partial(
        pl.kernel,
        mesh=plsc.VectorSubcoreMesh(core_axis_name="c", subcore_axis_name="s"),
        compiler_params=pltpu.CompilerParams(needs_layout_passes=False),
        out_type=[
            jax.ShapeDtypeStruct((B,), jnp.float32),
            jax.ShapeDtypeStruct((B,), jnp.float32),
        ],
        scratch_types=[
            pltpu.VMEM((B,), jnp.int32),
            pltpu.VMEM((RPW, IDX_COLS), jnp.int32),
            pltpu.VMEM((RPW * IDX_COLS,), jnp.int32),
            pltpu.VMEM((RPW * IDX_COLS,), jnp.int32),
            pltpu.VMEM((RPW * IDX_COLS,), jnp.float32),
            pltpu.VMEM((RPW * IDX_COLS,), jnp.float32),
            pltpu.VMEM((RPW * IDX_COLS,), jnp.float32),
            pltpu.VMEM((RPW,), jnp.float32),
            pltpu.VMEM((RPW,), jnp.float32),
            pltpu.SemaphoreType.DMA,
        ],
    )(_sc_body)


def kernel(image_features, text_features, logit_scale, img_index):
    scale2d = jnp.asarray(logit_scale, jnp.float32).reshape(1, 1)
    logits, lse_row, lse_col, idx = _dense_call(
        scale2d, image_features, text_features, text_features)
    a_img, a_txt = _sc_call()(logits.reshape(-1), idx, img_index)
    image_loss = jnp.mean(lse_row[:, 0] - a_img)
    text_loss = jnp.mean(lse_col[0, :] - a_txt)
    return (image_loss, text_loss, logits)


# speedup vs baseline: 1.7552x; 1.0225x over previous
import functools

import jax
import jax.numpy as jnp
from jax import lax
from jax.experimental import pallas as pl
from jax.experimental.pallas import tpu as pltpu
from jax.experimental.pallas import tpu_sc as plsc

B = 4096
D = 1024
BR = 512
CT = 2048
NRB = B // BR
NCT = B // CT
TOPK = 10
IDX_COLS = 16
NEG = -3.0e38

NC = 2
NS = 16
NW = NC * NS
RPW = B // NW


def _dense_body(scale_ref, img_ref, txt_r_ref, txt_c_ref,
                l_ref, lse_row_ref, lse_col_ref, idx_ref,
                sim_scr, rmax_scr, rsum_scr, cmax_scr, csum_scr):
    rb = pl.program_id(0)
    ct = pl.program_id(1)
    scale = scale_ref[0, 0]

    img = img_ref[...]
    txt_r = txt_r_ref[...]
    txt_c = txt_c_ref[...]

    l_tile = scale * lax.dot_general(
        img.astype(jnp.bfloat16), txt_c.astype(jnp.bfloat16),
        (((1,), (1,)), ((), ())),
        preferred_element_type=jnp.float32)
    l_ref[...] = l_tile

    inv_r = 1.0 / jnp.maximum(
        jnp.sqrt(jnp.sum(txt_r * txt_r, axis=1, keepdims=True)), 1e-12)
    inv_c = 1.0 / jnp.maximum(
        jnp.sqrt(jnp.sum(txt_c * txt_c, axis=1, keepdims=True)), 1e-12)
    sim_tile = lax.dot_general(
        (txt_r * inv_r).astype(jnp.bfloat16),
        (txt_c * inv_c).astype(jnp.bfloat16),
        (((1,), (1,)), ((), ())),
        preferred_element_type=jnp.float32)

    sim_scr[:, pl.ds(ct * CT, CT)] = sim_tile

    t_rmax = jnp.max(l_tile, axis=1, keepdims=True)

    @pl.when(ct == 0)
    def _():
        rmax_scr[:, 0:1] = t_rmax
        rsum_scr[:, 0:1] = jnp.sum(jnp.exp(l_tile - t_rmax), axis=1,
                                   keepdims=True)

    @pl.when(ct > 0)
    def _():
        om = rmax_scr[:, 0:1]
        nm = jnp.maximum(om, t_rmax)
        rsum_scr[:, 0:1] = (rsum_scr[:, 0:1] * jnp.exp(om - nm)
                            + jnp.sum(jnp.exp(l_tile - nm), axis=1,
                                      keepdims=True))
        rmax_scr[:, 0:1] = nm

    t_cmax = jnp.max(l_tile, axis=0, keepdims=True)

    @pl.when(rb == 0)
    def _():
        cmax_scr[0:1, pl.ds(ct * CT, CT)] = t_cmax
        csum_scr[0:1, pl.ds(ct * CT, CT)] = jnp.sum(
            jnp.exp(l_tile - t_cmax), axis=0, keepdims=True)

    @pl.when(rb > 0)
    def _():
        om = cmax_scr[0:1, pl.ds(ct * CT, CT)]
        nm = jnp.maximum(om, t_cmax)
        csum_scr[0:1, pl.ds(ct * CT, CT)] = (
            csum_scr[0:1, pl.ds(ct * CT, CT)] * jnp.exp(om - nm)
            + jnp.sum(jnp.exp(l_tile - nm), axis=0, keepdims=True))
        cmax_scr[0:1, pl.ds(ct * CT, CT)] = nm

    @pl.when(ct == NCT - 1)
    def _():
        col_iota = lax.broadcasted_iota(jnp.int32, (BR, B), 1)
        row_ids = rb * BR + lax.broadcasted_iota(jnp.int32, (BR, B), 0)
        w = jnp.where(col_iota == row_ids, 0.0, sim_scr[...])
        u = lax.bitcast_convert_type(w, jnp.int32)
        ki = jnp.where(w < 0, u ^ jnp.int32(0x7FFFFFFF), u)
        mk = (ki & jnp.int32(-4096)) | ((B - 1) - col_iota)
        for m in range(TOPK):
            mx = jnp.max(mk, axis=1, keepdims=True)
            idx_ref[:, m:m + 1] = (B - 1) - (mx & jnp.int32(4095))
            mk = jnp.where(mk == mx, jnp.int32(-2147483648), mk)
        own = rb * BR + lax.broadcasted_iota(jnp.int32, (BR, 1), 0)
        for m in range(TOPK, IDX_COLS):
            idx_ref[:, m:m + 1] = own
        lse_row_ref[...] = rmax_scr[:, 0:1] + jnp.log(rsum_scr[:, 0:1])

    @pl.when((rb == NRB - 1) & (ct == NCT - 1))
    def _():
        lse_col_ref[...] = cmax_scr[0:1, :] + jnp.log(csum_scr[0:1, :])


_dense_call = pl.pallas_call(
    _dense_body,
    grid=(NRB, NCT),
    in_specs=[
        pl.BlockSpec((1, 1), lambda rb, ct: (0, 0),
                     memory_space=pltpu.SMEM),
        pl.BlockSpec((BR, D), lambda rb, ct: (rb, 0)),
        pl.BlockSpec((BR, D), lambda rb, ct: (rb, 0)),
        pl.BlockSpec((CT, D), lambda rb, ct: (ct, 0)),
    ],
    out_specs=[
        pl.BlockSpec((BR, CT), lambda rb, ct: (rb, ct)),
        pl.BlockSpec((BR, 1), lambda rb, ct: (rb, 0)),
        pl.BlockSpec((1, B), lambda rb, ct: (0, 0)),
        pl.BlockSpec((BR, IDX_COLS), lambda rb, ct: (rb, 0)),
    ],
    out_shape=[
        jax.ShapeDtypeStruct((B, B), jnp.float32),
        jax.ShapeDtypeStruct((B, 1), jnp.float32),
        jax.ShapeDtypeStruct((1, B), jnp.float32),
        jax.ShapeDtypeStruct((B, IDX_COLS), jnp.int32),
    ],
    scratch_shapes=[
        pltpu.VMEM((BR, B), jnp.float32),
        pltpu.VMEM((BR, 128), jnp.float32),
        pltpu.VMEM((BR, 128), jnp.float32),
        pltpu.VMEM((8, B), jnp.float32),
        pltpu.VMEM((8, B), jnp.float32),
    ],
    compiler_params=pltpu.CompilerParams(
        dimension_semantics=("arbitrary", "arbitrary")),
)


def _sc_body(lflat_hbm, idx_hbm, cls_hbm, aimg_hbm, atxt_hbm,
             cls_v, idx_v, fimg_v, ftxt_v, vimg_v, vtxt_v, wgt_v,
             oimg_v, otxt_v, sem):
    wid = lax.axis_index("s") * NC + lax.axis_index("c")
    base = wid * RPW
    pltpu.sync_copy(cls_hbm, cls_v)
    pltpu.sync_copy(idx_hbm.at[pl.ds(base, RPW)], idx_v)

    lane = lax.iota(jnp.int32, 16)

    def build(jl, carry):
        j = base + jl
        jv = jnp.broadcast_to(j, (16,))
        iv = idx_v[jl, :]
        iv = jnp.where(lane >= TOPK, jv, iv)
        cls_i = plsc.load_gather(cls_v, [iv])
        cls_j = plsc.load_gather(cls_v, [jv])
        keep = (cls_i == cls_j) & (iv != jv) & (lane < TOPK)
        keep = keep | (lane == TOPK)
        wgt_v[pl.ds(jl * IDX_COLS, IDX_COLS)] = jnp.where(keep, 1.0, 0.0)
        fimg_v[pl.ds(jl * IDX_COLS, IDX_COLS)] = jv * B + iv
        ftxt_v[pl.ds(jl * IDX_COLS, IDX_COLS)] = iv * B + jv
        return carry

    lax.fori_loop(0, RPW, build, 0)

    copies = []
    for c in range(RPW * IDX_COLS // 128):
        sl = pl.ds(c * 128, 128)
        copies.append(pltpu.async_copy(
            lflat_hbm.at[fimg_v.at[sl]], vimg_v.at[sl], sem))
        copies.append(pltpu.async_copy(
            lflat_hbm.at[ftxt_v.at[sl]], vtxt_v.at[sl], sem))
    for cp in copies:
        cp.wait()

    def accum(c, carry):
        rows16 = c * IDX_COLS + lane
        gdiag = rows16 * IDX_COLS + TOPK
        acc_i = plsc.load_gather(vimg_v, [gdiag])
        acc_t = plsc.load_gather(vtxt_v, [gdiag])
        s = jnp.full((16,), 1.0, jnp.float32)
        for m in range(TOPK):
            g = rows16 * IDX_COLS + m
            w = plsc.load_gather(wgt_v, [g])
            s = s + w
            acc_i = acc_i + w * plsc.load_gather(vimg_v, [g])
            acc_t = acc_t + w * plsc.load_gather(vtxt_v, [g])
        oimg_v[pl.ds(c * IDX_COLS, IDX_COLS)] = acc_i / s
        otxt_v[pl.ds(c * IDX_COLS, IDX_COLS)] = acc_t / s
        return carry

    lax.fori_loop(0, RPW // IDX_COLS, accum, 0)

    pltpu.sync_copy(oimg_v, aimg_hbm.at[pl.ds(base, RPW)])
    pltpu.sync_copy(otxt_v, atxt_hbm.at[pl.ds(base, RPW)])


@functools.cache
def _sc_call():
    return functools.partial(
        pl.kernel,
        mesh=plsc.VectorSubcoreMesh(core_axis_name="c", subcore_axis_name="s"),
        compiler_params=pltpu.CompilerParams(needs_layout_passes=False),
        out_type=[
            jax.ShapeDtypeStruct((B,), jnp.float32),
            jax.ShapeDtypeStruct((B,), jnp.float32),
        ],
        scratch_types=[
            pltpu.VMEM((B,), jnp.int32),
            pltpu.VMEM((RPW, IDX_COLS), jnp.int32),
            pltpu.VMEM((RPW * IDX_COLS,), jnp.int32),
            pltpu.VMEM((RPW * IDX_COLS,), jnp.int32),
            pltpu.VMEM((RPW * IDX_COLS,), jnp.float32),
            pltpu.VMEM((RPW * IDX_COLS,), jnp.float32),
            pltpu.VMEM((RPW * IDX_COLS,), jnp.float32),
            pltpu.VMEM((RPW,), jnp.float32),
            pltpu.VMEM((RPW,), jnp.float32),
            pltpu.SemaphoreType.DMA,
        ],
    )(_sc_body)


def kernel(image_features, text_features, logit_scale, img_index):
    scale2d = jnp.asarray(logit_scale, jnp.float32).reshape(1, 1)
    logits, lse_row, lse_col, idx = _dense_call(
        scale2d, image_features, text_features, text_features)
    a_img, a_txt = _sc_call()(logits.reshape(-1), idx, img_index)
    image_loss = jnp.mean(lse_row[:, 0] - a_img)
    text_loss = jnp.mean(lse_col[0, :] - a_txt)
    return (image_loss, text_loss, logits)


# speedup vs baseline: 1.8059x; 1.0289x over previous
import functools

import jax
import jax.numpy as jnp
from jax import lax
from jax.experimental import pallas as pl
from jax.experimental.pallas import tpu as pltpu
from jax.experimental.pallas import tpu_sc as plsc

B = 4096
D = 1024
BR = 512
CT = 1024
NRB = B // BR
NCT = B // CT
TOPK = 10
IDX_COLS = 16
NEG = -3.0e38

NC = 2
NS = 16
NW = NC * NS
RPW = B // NW


def _dense_body(scale_ref, img_ref, txt_r_ref, txt_c_ref,
                l_ref, lse_row_ref, lse_col_ref, idx_ref,
                sim_scr, rmax_scr, rsum_scr, cmax_scr, csum_scr):
    rb = pl.program_id(0)
    ct = pl.program_id(1)
    scale = scale_ref[0, 0]

    img = img_ref[...]
    txt_r = txt_r_ref[...]
    txt_c = txt_c_ref[...]

    l_tile = scale * lax.dot_general(
        img.astype(jnp.bfloat16), txt_c.astype(jnp.bfloat16),
        (((1,), (1,)), ((), ())),
        preferred_element_type=jnp.float32)
    l_ref[...] = l_tile

    inv_r = 1.0 / jnp.maximum(
        jnp.sqrt(jnp.sum(txt_r * txt_r, axis=1, keepdims=True)), 1e-12)
    inv_c = 1.0 / jnp.maximum(
        jnp.sqrt(jnp.sum(txt_c * txt_c, axis=1, keepdims=True)), 1e-12)
    sim_tile = lax.dot_general(
        (txt_r * inv_r).astype(jnp.bfloat16),
        (txt_c * inv_c).astype(jnp.bfloat16),
        (((1,), (1,)), ((), ())),
        preferred_element_type=jnp.float32)

    sim_scr[:, pl.ds(ct * CT, CT)] = sim_tile

    t_rmax = jnp.max(l_tile, axis=1, keepdims=True)

    @pl.when(ct == 0)
    def _():
        rmax_scr[:, 0:1] = t_rmax
        rsum_scr[:, 0:1] = jnp.sum(jnp.exp(l_tile - t_rmax), axis=1,
                                   keepdims=True)

    @pl.when(ct > 0)
    def _():
        om = rmax_scr[:, 0:1]
        nm = jnp.maximum(om, t_rmax)
        rsum_scr[:, 0:1] = (rsum_scr[:, 0:1] * jnp.exp(om - nm)
                            + jnp.sum(jnp.exp(l_tile - nm), axis=1,
                                      keepdims=True))
        rmax_scr[:, 0:1] = nm

    t_cmax = jnp.max(l_tile, axis=0, keepdims=True)

    @pl.when(rb == 0)
    def _():
        cmax_scr[0:1, pl.ds(ct * CT, CT)] = t_cmax
        csum_scr[0:1, pl.ds(ct * CT, CT)] = jnp.sum(
            jnp.exp(l_tile - t_cmax), axis=0, keepdims=True)

    @pl.when(rb > 0)
    def _():
        om = cmax_scr[0:1, pl.ds(ct * CT, CT)]
        nm = jnp.maximum(om, t_cmax)
        csum_scr[0:1, pl.ds(ct * CT, CT)] = (
            csum_scr[0:1, pl.ds(ct * CT, CT)] * jnp.exp(om - nm)
            + jnp.sum(jnp.exp(l_tile - nm), axis=0, keepdims=True))
        cmax_scr[0:1, pl.ds(ct * CT, CT)] = nm

    @pl.when(ct == NCT - 1)
    def _():
        col_iota = lax.broadcasted_iota(jnp.int32, (BR, B), 1)
        row_ids = rb * BR + lax.broadcasted_iota(jnp.int32, (BR, B), 0)
        w = jnp.where(col_iota == row_ids, 0.0, sim_scr[...])
        u = lax.bitcast_convert_type(w, jnp.int32)
        ki = jnp.where(w < 0, u ^ jnp.int32(0x7FFFFFFF), u)
        mk = (ki & jnp.int32(-4096)) | ((B - 1) - col_iota)
        ka = mk[:, :B // 2]
        kb = mk[:, B // 2:]
        f = jnp.maximum(ka, kb)
        l = jnp.minimum(ka, kb)
        kmin = jnp.int32(-2147483648)
        for m in range(TOPK):
            mx = jnp.max(f, axis=1, keepdims=True)
            idx_ref[:, m:m + 1] = (B - 1) - (mx & jnp.int32(4095))
            hit = f == mx
            f = jnp.where(hit, l, f)
            l = jnp.where(hit, kmin, l)
        own = rb * BR + lax.broadcasted_iota(jnp.int32, (BR, 1), 0)
        for m in range(TOPK, IDX_COLS):
            idx_ref[:, m:m + 1] = own
        lse_row_ref[...] = rmax_scr[:, 0:1] + jnp.log(rsum_scr[:, 0:1])

    @pl.when((rb == NRB - 1) & (ct == NCT - 1))
    def _():
        lse_col_ref[...] = cmax_scr[0:1, :] + jnp.log(csum_scr[0:1, :])


_dense_call = pl.pallas_call(
    _dense_body,
    grid=(NRB, NCT),
    in_specs=[
        pl.BlockSpec((1, 1), lambda rb, ct: (0, 0),
                     memory_space=pltpu.SMEM),
        pl.BlockSpec((BR, D), lambda rb, ct: (rb, 0)),
        pl.BlockSpec((BR, D), lambda rb, ct: (rb, 0)),
        pl.BlockSpec((CT, D), lambda rb, ct: (ct, 0)),
    ],
    out_specs=[
        pl.BlockSpec((BR, CT), lambda rb, ct: (rb, ct)),
        pl.BlockSpec((BR, 1), lambda rb, ct: (rb, 0)),
        pl.BlockSpec((1, B), lambda rb, ct: (0, 0)),
        pl.BlockSpec((BR, IDX_COLS), lambda rb, ct: (rb, 0)),
    ],
    out_shape=[
        jax.ShapeDtypeStruct((B, B), jnp.float32),
        jax.ShapeDtypeStruct((B, 1), jnp.float32),
        jax.ShapeDtypeStruct((1, B), jnp.float32),
        jax.ShapeDtypeStruct((B, IDX_COLS), jnp.int32),
    ],
    scratch_shapes=[
        pltpu.VMEM((BR, B), jnp.float32),
        pltpu.VMEM((BR, 128), jnp.float32),
        pltpu.VMEM((BR, 128), jnp.float32),
        pltpu.VMEM((8, B), jnp.float32),
        pltpu.VMEM((8, B), jnp.float32),
    ],
    compiler_params=pltpu.CompilerParams(
        dimension_semantics=("arbitrary", "arbitrary")),
)


def _sc_body(lflat_hbm, idx_hbm, cls_hbm, aimg_hbm, atxt_hbm,
             cls_v, idx_v, fimg_v, ftxt_v, vimg_v, vtxt_v, wgt_v,
             oimg_v, otxt_v, sem):
    wid = lax.axis_index("s") * NC + lax.axis_index("c")
    base = wid * RPW
    pltpu.sync_copy(cls_hbm, cls_v)
    pltpu.sync_copy(idx_hbm.at[pl.ds(base, RPW)], idx_v)

    lane = lax.iota(jnp.int32, 16)

    def build(jl, carry):
        j = base + jl
        jv = jnp.broadcast_to(j, (16,))
        iv = idx_v[jl, :]
        iv = jnp.where(lane >= TOPK, jv, iv)
        cls_i = plsc.load_gather(cls_v, [iv])
        cls_j = plsc.load_gather(cls_v, [jv])
        keep = (cls_i == cls_j) & (iv != jv) & (lane < TOPK)
        keep = keep | (lane == TOPK)
        wgt_v[pl.ds(jl * IDX_COLS, IDX_COLS)] = jnp.where(keep, 1.0, 0.0)
        fimg_v[pl.ds(jl * IDX_COLS, IDX_COLS)] = jv * B + iv
        ftxt_v[pl.ds(jl * IDX_COLS, IDX_COLS)] = iv * B + jv
        return carry

    lax.fori_loop(0, RPW, build, 0)

    copies = []
    for c in range(RPW * IDX_COLS // 128):
        sl = pl.ds(c * 128, 128)
        copies.append(pltpu.async_copy(
            lflat_hbm.at[fimg_v.at[sl]], vimg_v.at[sl], sem))
        copies.append(pltpu.async_copy(
            lflat_hbm.at[ftxt_v.at[sl]], vtxt_v.at[sl], sem))
    for cp in copies:
        cp.wait()

    def accum(c, carry):
        rows16 = c * IDX_COLS + lane
        gdiag = rows16 * IDX_COLS + TOPK
        acc_i = plsc.load_gather(vimg_v, [gdiag])
        acc_t = plsc.load_gather(vtxt_v, [gdiag])
        s = jnp.full((16,), 1.0, jnp.float32)
        for m in range(TOPK):
            g = rows16 * IDX_COLS + m
            w = plsc.load_gather(wgt_v, [g])
            s = s + w
            acc_i = acc_i + w * plsc.load_gather(vimg_v, [g])
            acc_t = acc_t + w * plsc.load_gather(vtxt_v, [g])
        oimg_v[pl.ds(c * IDX_COLS, IDX_COLS)] = acc_i / s
        otxt_v[pl.ds(c * IDX_COLS, IDX_COLS)] = acc_t / s
        return carry

    lax.fori_loop(0, RPW // IDX_COLS, accum, 0)

    pltpu.sync_copy(oimg_v, aimg_hbm.at[pl.ds(base, RPW)])
    pltpu.sync_copy(otxt_v, atxt_hbm.at[pl.ds(base, RPW)])


@functools.cache
def _sc_call():
    return functools.partial(
        pl.kernel,
        mesh=plsc.VectorSubcoreMesh(core_axis_name="c", subcore_axis_name="s"),
        compiler_params=pltpu.CompilerParams(needs_layout_passes=False),
        out_type=[
            jax.ShapeDtypeStruct((B,), jnp.float32),
            jax.ShapeDtypeStruct((B,), jnp.float32),
        ],
        scratch_types=[
            pltpu.VMEM((B,), jnp.int32),
            pltpu.VMEM((RPW, IDX_COLS), jnp.int32),
            pltpu.VMEM((RPW * IDX_COLS,), jnp.int32),
            pltpu.VMEM((RPW * IDX_COLS,), jnp.int32),
            pltpu.VMEM((RPW * IDX_COLS,), jnp.float32),
            pltpu.VMEM((RPW * IDX_COLS,), jnp.float32),
            pltpu.VMEM((RPW * IDX_COLS,), jnp.float32),
            pltpu.VMEM((RPW,), jnp.float32),
            pltpu.VMEM((RPW,), jnp.float32),
            pltpu.SemaphoreType.DMA,
        ],
    )(_sc_body)


def kernel(image_features, text_features, logit_scale, img_index):
    scale2d = jnp.asarray(logit_scale, jnp.float32).reshape(1, 1)
    logits, lse_row, lse_col, idx = _dense_call(
        scale2d, image_features, text_features, text_features)
    a_img, a_txt = _sc_call()(logits.reshape(-1), idx, img_index)
    image_loss = jnp.mean(lse_row[:, 0] - a_img)
    text_loss = jnp.mean(lse_col[0, :] - a_txt)
    return (image_loss, text_loss, logits)
